# Initial kernel scaffold; baseline (speedup 1.0000x reference)
import jax
import jax.numpy as jnp
from jax.experimental import pallas as pl


def kernel(x, edge_index, W_l0, b_l0, W_r0, W_l1, b_l1, W_r1):
    raise NotImplementedError("write your pallas kernel here")


# speedup vs baseline: 5.1847x; 5.1847x over previous
import functools

import jax
import jax.numpy as jnp
from jax import lax
from jax.experimental import pallas as pl
from jax.experimental.pallas import tpu as pltpu
from jax.experimental.pallas import tpu_sc as plsc

N_NODES = 10000
N_EDGES = 320000
D = 128
CHUNK = 128
N_CHUNKS = N_EDGES // CHUNK
NC = 2
NS = 16
NW = NC * NS
ROWS_PER_TILE = N_NODES // NS

_MESH = plsc.VectorSubcoreMesh(core_axis_name="c", subcore_axis_name="s")
_OUT_SEG = [jax.ShapeDtypeStruct((NC, NS, ROWS_PER_TILE, D), jnp.float32)]
_SEG_SCRATCH = dict(
    srcb=pltpu.VMEM((CHUNK,), jnp.int32),
    dstb=pltpu.VMEM((CHUNK,), jnp.int32),
    rows=pltpu.VMEM((CHUNK, D), jnp.float32),
    acc_sh=pltpu.VMEM_SHARED((N_NODES, D), jnp.float32),
    sem=pltpu.SemaphoreType.DMA,
)


@functools.partial(pl.kernel, mesh=_MESH, out_type=_OUT_SEG,
                   scratch_types=_SEG_SCRATCH)
def _sc_segsum(x_hbm, src_hbm, dst_hbm, z128_hbm, out_acc,
               srcb, dstb, rows, acc_sh, sem):
    c = lax.axis_index("c")
    s = lax.axis_index("s")
    wid = s * NC + c
    row0 = s * ROWS_PER_TILE

    pltpu.sync_copy(z128_hbm, acc_sh.at[pl.ds(row0, ROWS_PER_TILE)])
    plsc.subcore_barrier()

    n_my = (N_CHUNKS - wid + NW - 1) // NW

    def body(i, carry):
        base = (wid + i * NW) * CHUNK
        pltpu.sync_copy(src_hbm.at[pl.ds(base, CHUNK)], srcb)
        pltpu.sync_copy(dst_hbm.at[pl.ds(base, CHUNK)], dstb)
        pltpu.async_copy(x_hbm.at[srcb], rows, sem).wait()
        pltpu.sync_copy(rows, acc_sh.at[dstb], add=True)
        return carry

    lax.fori_loop(0, n_my, body, 0)
    plsc.subcore_barrier()

    pltpu.sync_copy(acc_sh.at[pl.ds(row0, ROWS_PER_TILE)], out_acc.at[c, s])


@functools.partial(pl.kernel, mesh=_MESH, out_type=_OUT_SEG,
                   scratch_types=dict(
                       dstb=pltpu.VMEM((CHUNK,), jnp.int32),
                       onesb=pltpu.VMEM((CHUNK, D), jnp.float32),
                       acc_sh=pltpu.VMEM_SHARED((N_NODES, D), jnp.float32),
                   ))
def _sc_counts(dst_hbm, z128_hbm, ones_hbm, out_cnt, dstb, onesb, acc_sh):
    c = lax.axis_index("c")
    s = lax.axis_index("s")
    wid = s * NC + c
    row0 = s * ROWS_PER_TILE

    pltpu.sync_copy(z128_hbm, acc_sh.at[pl.ds(row0, ROWS_PER_TILE)])
    pltpu.sync_copy(ones_hbm, onesb)
    plsc.subcore_barrier()

    n_my = (N_CHUNKS - wid + NW - 1) // NW

    def body(i, carry):
        base = (wid + i * NW) * CHUNK
        pltpu.sync_copy(dst_hbm.at[pl.ds(base, CHUNK)], dstb)
        pltpu.sync_copy(onesb, acc_sh.at[dstb], add=True)
        return carry

    lax.fori_loop(0, n_my, body, 0)
    plsc.subcore_barrier()

    pltpu.sync_copy(acc_sh.at[pl.ds(row0, ROWS_PER_TILE)], out_cnt.at[c, s])


def _tc_combine_body(relu, acc_ref, cnt_ref, x_ref, wl_ref, wr_ref, b_ref,
                     o_ref):
    a = acc_ref[0] + acc_ref[1]
    cnt = cnt_ref[0, :, 0:1] + cnt_ref[1, :, 0:1]
    agg = a * (1.0 / jnp.maximum(cnt, 1.0))
    y = (jnp.dot(agg, wl_ref[...], preferred_element_type=jnp.float32)
         + jnp.dot(x_ref[...], wr_ref[...], preferred_element_type=jnp.float32)
         + b_ref[...])
    if relu:
        y = jnp.maximum(y, 0.0)
    o_ref[...] = y


def _tc_combine(acc, cnt, x, W_l, W_r, b, relu):
    R = 1000
    return pl.pallas_call(
        functools.partial(_tc_combine_body, relu),
        grid=(N_NODES // R,),
        in_specs=[
            pl.BlockSpec((NC, R, D), lambda r: (0, r, 0)),
            pl.BlockSpec((NC, R, D), lambda r: (0, r, 0)),
            pl.BlockSpec((R, D), lambda r: (r, 0)),
            pl.BlockSpec((D, D), lambda r: (0, 0)),
            pl.BlockSpec((D, D), lambda r: (0, 0)),
            pl.BlockSpec((1, D), lambda r: (0, 0)),
        ],
        out_specs=pl.BlockSpec((R, D), lambda r: (r, 0)),
        out_shape=jax.ShapeDtypeStruct((N_NODES, D), jnp.float32),
    )(acc, cnt, x, W_l, W_r, b)


def kernel(x, edge_index, W_l0, b_l0, W_r0, W_l1, b_l1, W_r1):
    src = edge_index[0].astype(jnp.int32)
    dst = edge_index[1].astype(jnp.int32)
    z128 = jnp.zeros((ROWS_PER_TILE, D), jnp.float32)
    ones128 = jnp.ones((CHUNK, D), jnp.float32)

    (cnt,) = _sc_counts(dst, z128, ones128)
    cnt = cnt.reshape(NC, N_NODES, D)
    (acc0,) = _sc_segsum(x, src, dst, z128)
    acc0 = acc0.reshape(NC, N_NODES, D)
    h = _tc_combine(acc0, cnt, x, W_l0, W_r0, b_l0.reshape(1, D), relu=True)
    (acc1,) = _sc_segsum(h, src, dst, z128)
    acc1 = acc1.reshape(NC, N_NODES, D)
    out = _tc_combine(acc1, cnt, h, W_l1, W_r1, b_l1.reshape(1, D), relu=False)
    return out
